# baseline (speedup 1.0000x reference)
import functools

import jax
import jax.numpy as jnp
from jax.experimental import pallas as pl
from jax.experimental.pallas import tpu as pltpu


def _conv_stage_kernel(x_ref, w_ref, b_ref, sc_ref, sh_ref,
                       y_ref, sum_ref, sq_ref, buf_ref,
                       *, H, W, K, affine):
    C = x_ref.shape[1]
    HW = H * W
    p = (K - 1) // 2
    pad = p * (W + 1)

    x = x_ref[0].astype(jnp.float32)
    if affine:
        x = x * sc_ref[...] + sh_ref[...]

    z = jnp.zeros((C, pad), jnp.bfloat16)
    buf_ref[:, 0:pad] = z
    buf_ref[:, pad:pad + HW] = x.astype(jnp.bfloat16)
    buf_ref[:, pad + HW:] = z

    col = jax.lax.broadcasted_iota(jnp.int32, (1, HW), 1) % W

    acc = None
    for kh in range(K):
        for kw in range(K):
            off = kh * W + kw
            t = buf_ref[:, off:off + HW]
            dw = kw - p
            if dw != 0:
                m = ((col + dw >= 0) & (col + dw < W)).astype(jnp.bfloat16)
                t = t * m
            d = jnp.dot(w_ref[kh * K + kw], t,
                        preferred_element_type=jnp.float32)
            acc = d if acc is None else acc + d

    y = jnp.maximum(acc + b_ref[...], 0.0)
    sum_ref[0] = jnp.sum(y, axis=1, keepdims=True)
    sq_ref[0] = jnp.sum(y * y, axis=1, keepdims=True)
    y_ref[0] = y.astype(y_ref.dtype)


def _conv_stage(x, wt, b, sc, sh, H, W, affine, out_dtype):
    N, C, HW = x.shape
    KK, Cout, _ = wt.shape
    K = int(round(KK ** 0.5))
    p = (K - 1) // 2
    L = HW + 2 * p * (W + 1)

    kern = functools.partial(_conv_stage_kernel, H=H, W=W, K=K, affine=affine)
    return pl.pallas_call(
        kern,
        grid=(N,),
        out_shape=(
            jax.ShapeDtypeStruct((N, Cout, HW), out_dtype),
            jax.ShapeDtypeStruct((N, Cout, 1), jnp.float32),
            jax.ShapeDtypeStruct((N, Cout, 1), jnp.float32),
        ),
        in_specs=[
            pl.BlockSpec((1, C, HW), lambda n: (n, 0, 0)),
            pl.BlockSpec((KK, Cout, C), lambda n: (0, 0, 0)),
            pl.BlockSpec((Cout, 1), lambda n: (0, 0)),
            pl.BlockSpec((C, 1), lambda n: (0, 0)),
            pl.BlockSpec((C, 1), lambda n: (0, 0)),
        ],
        out_specs=(
            pl.BlockSpec((1, Cout, HW), lambda n: (n, 0, 0)),
            pl.BlockSpec((1, Cout, 1), lambda n: (n, 0, 0)),
            pl.BlockSpec((1, Cout, 1), lambda n: (n, 0, 0)),
        ),
        scratch_shapes=[pltpu.VMEM((C, L), jnp.bfloat16)],
        compiler_params=pltpu.CompilerParams(
            dimension_semantics=("parallel",),
        ),
    )(x, wt, b, sc, sh)


def _affine_nchw_kernel(y_ref, sc_ref, sh_ref, o_ref):
    o_ref[0] = (y_ref[0].astype(jnp.float32) * sc_ref[...]
                + sh_ref[...]).astype(o_ref.dtype)


def _apply_affine(y, sc, sh, out_dtype):
    N, C, HW = y.shape
    return pl.pallas_call(
        _affine_nchw_kernel,
        grid=(N,),
        out_shape=jax.ShapeDtypeStruct((N, C, HW), out_dtype),
        in_specs=[
            pl.BlockSpec((1, C, HW), lambda n: (n, 0, 0)),
            pl.BlockSpec((C, 1), lambda n: (0, 0)),
            pl.BlockSpec((C, 1), lambda n: (0, 0)),
        ],
        out_specs=pl.BlockSpec((1, C, HW), lambda n: (n, 0, 0)),
        compiler_params=pltpu.CompilerParams(
            dimension_semantics=("parallel",),
        ),
    )(y, sc, sh)


def _bn_affine(part_sum, part_sq, gamma, beta, count, eps):
    s = jnp.sum(part_sum[:, :, 0], axis=0)
    q = jnp.sum(part_sq[:, :, 0], axis=0)
    mean = s / count
    var = jnp.maximum(q / count - mean * mean, 0.0)
    inv = jax.lax.rsqrt(var + eps)
    scale = gamma.astype(jnp.float32) * inv
    shift = beta.astype(jnp.float32) - mean * scale
    C = scale.shape[0]
    return scale.reshape(C, 1), shift.reshape(C, 1)


def kernel(x, w1, b1, g1, be1, w2, b2, g2, be2):
    N, Cin, H, W = x.shape
    K = w1.shape[0]
    C1 = w1.shape[3]
    C2 = w2.shape[3]
    eps = 1e-5

    xf = x.reshape(N, Cin, H * W)
    w1t = jnp.transpose(w1.astype(jnp.float32).reshape(K * K, Cin, C1),
                        (0, 2, 1)).astype(jnp.bfloat16)
    w2t = jnp.transpose(w2.astype(jnp.float32).reshape(K * K, C1, C2),
                        (0, 2, 1)).astype(jnp.bfloat16)
    b1c = b1.astype(jnp.float32).reshape(C1, 1)
    b2c = b2.astype(jnp.float32).reshape(C2, 1)
    one = jnp.ones((Cin, 1), jnp.float32)
    zero = jnp.zeros((Cin, 1), jnp.float32)

    y1, s1, q1 = _conv_stage(xf, w1t, b1c, one, zero, H, W,
                             affine=False, out_dtype=jnp.bfloat16)
    sc1, sh1 = _bn_affine(s1, q1, g1, be1, N * H * W, eps)

    y2, s2, q2 = _conv_stage(y1, w2t, b2c, sc1, sh1, H, W,
                             affine=True, out_dtype=jnp.bfloat16)
    sc2, sh2 = _bn_affine(s2, q2, g2, be2, N * H * W, eps)

    out = _apply_affine(y2, sc2, sh2, x.dtype)
    return out.reshape(N, C2, H, W)


# speedup vs baseline: 1.2273x; 1.2273x over previous
import functools

import jax
import jax.numpy as jnp
from jax.experimental import pallas as pl
from jax.experimental.pallas import tpu as pltpu


def _conv_stage_kernel(x_ref, w_ref, b_ref, sc_ref, sh_ref,
                       y_ref, sum_ref, sq_ref, xp_ref,
                       *, H, W, K, transpose_in, affine):
    HW = H * W
    p = (K - 1) // 2
    Hp = H + 2 * p
    Wp = W + 2 * p

    if transpose_in:
        C = x_ref.shape[1]
        x = jnp.transpose(x_ref[0]).astype(jnp.float32)
    else:
        C = x_ref.shape[2]
        x = x_ref[0].astype(jnp.float32)
    if affine:
        x = x * sc_ref[...] + sh_ref[...]
    xb = x.astype(jnp.bfloat16)

    if p > 0:
        xp_ref[0:p, :, :] = jnp.zeros((p, Wp, C), jnp.bfloat16)
        xp_ref[p + H:, :, :] = jnp.zeros((p, Wp, C), jnp.bfloat16)
        xp_ref[p:p + H, 0:p, :] = jnp.zeros((H, p, C), jnp.bfloat16)
        xp_ref[p:p + H, p + W:, :] = jnp.zeros((H, p, C), jnp.bfloat16)
    xp_ref[p:p + H, p:p + W, :] = xb.reshape(H, W, C)

    acc = None
    for kh in range(K):
        for kw in range(K):
            tap = xp_ref[kh:kh + H, kw:kw + W, :]
            d = jnp.dot(tap.reshape(HW, C), w_ref[kh * K + kw],
                        preferred_element_type=jnp.float32)
            acc = d if acc is None else acc + d

    y = jnp.maximum(acc + b_ref[...], 0.0)
    sum_ref[0] = jnp.sum(y, axis=0, keepdims=True)
    sq_ref[0] = jnp.sum(y * y, axis=0, keepdims=True)
    y_ref[0] = y.astype(y_ref.dtype)


def _conv_stage(x, w3, b, sc, sh, H, W, transpose_in, affine):
    N = x.shape[0]
    KK, C, Cout = w3.shape
    K = int(round(KK ** 0.5))
    p = (K - 1) // 2
    HW = H * W

    if transpose_in:
        x_spec = pl.BlockSpec((1, C, HW), lambda n: (n, 0, 0))
    else:
        x_spec = pl.BlockSpec((1, HW, C), lambda n: (n, 0, 0))

    kern = functools.partial(_conv_stage_kernel, H=H, W=W, K=K,
                             transpose_in=transpose_in, affine=affine)
    return pl.pallas_call(
        kern,
        grid=(N,),
        out_shape=(
            jax.ShapeDtypeStruct((N, HW, Cout), jnp.bfloat16),
            jax.ShapeDtypeStruct((N, 1, Cout), jnp.float32),
            jax.ShapeDtypeStruct((N, 1, Cout), jnp.float32),
        ),
        in_specs=[
            x_spec,
            pl.BlockSpec((KK, C, Cout), lambda n: (0, 0, 0)),
            pl.BlockSpec((1, Cout), lambda n: (0, 0)),
            pl.BlockSpec((1, C), lambda n: (0, 0)),
            pl.BlockSpec((1, C), lambda n: (0, 0)),
        ],
        out_specs=(
            pl.BlockSpec((1, HW, Cout), lambda n: (n, 0, 0)),
            pl.BlockSpec((1, 1, Cout), lambda n: (n, 0, 0)),
            pl.BlockSpec((1, 1, Cout), lambda n: (n, 0, 0)),
        ),
        scratch_shapes=[
            pltpu.VMEM((H + 2 * p, W + 2 * p, C), jnp.bfloat16),
        ],
        compiler_params=pltpu.CompilerParams(
            dimension_semantics=("parallel",),
        ),
    )(x, w3, b, sc, sh)


def _affine_nchw_kernel(y_ref, sc_ref, sh_ref, o_ref):
    y = y_ref[0].astype(jnp.float32) * sc_ref[...] + sh_ref[...]
    o_ref[0] = jnp.transpose(y).astype(o_ref.dtype)


def _apply_affine_nchw(y, sc, sh, out_dtype):
    N, HW, C = y.shape
    return pl.pallas_call(
        _affine_nchw_kernel,
        grid=(N,),
        out_shape=jax.ShapeDtypeStruct((N, C, HW), out_dtype),
        in_specs=[
            pl.BlockSpec((1, HW, C), lambda n: (n, 0, 0)),
            pl.BlockSpec((1, C), lambda n: (0, 0)),
            pl.BlockSpec((1, C), lambda n: (0, 0)),
        ],
        out_specs=pl.BlockSpec((1, C, HW), lambda n: (n, 0, 0)),
        compiler_params=pltpu.CompilerParams(
            dimension_semantics=("parallel",),
        ),
    )(y, sc, sh)


def _bn_affine(part_sum, part_sq, gamma, beta, count, eps):
    s = jnp.sum(part_sum[:, 0, :], axis=0)
    q = jnp.sum(part_sq[:, 0, :], axis=0)
    mean = s / count
    var = jnp.maximum(q / count - mean * mean, 0.0)
    inv = jax.lax.rsqrt(var + eps)
    scale = gamma.astype(jnp.float32) * inv
    shift = beta.astype(jnp.float32) - mean * scale
    C = scale.shape[0]
    return scale.reshape(1, C), shift.reshape(1, C)


def kernel(x, w1, b1, g1, be1, w2, b2, g2, be2):
    N, Cin, H, W = x.shape
    K = w1.shape[0]
    C1 = w1.shape[3]
    C2 = w2.shape[3]
    eps = 1e-5

    xf = x.reshape(N, Cin, H * W)
    w1b = w1.astype(jnp.bfloat16).reshape(K * K, Cin, C1)
    w2b = w2.astype(jnp.bfloat16).reshape(K * K, C1, C2)
    b1c = b1.astype(jnp.float32).reshape(1, C1)
    b2c = b2.astype(jnp.float32).reshape(1, C2)
    one = jnp.ones((1, Cin), jnp.float32)
    zero = jnp.zeros((1, Cin), jnp.float32)

    y1, s1, q1 = _conv_stage(xf, w1b, b1c, one, zero, H, W,
                             transpose_in=True, affine=False)
    sc1, sh1 = _bn_affine(s1, q1, g1, be1, N * H * W, eps)

    y2, s2, q2 = _conv_stage(y1, w2b, b2c, sc1, sh1, H, W,
                             transpose_in=False, affine=True)
    sc2, sh2 = _bn_affine(s2, q2, g2, be2, N * H * W, eps)

    out = _apply_affine_nchw(y2, sc2, sh2, x.dtype)
    return out.reshape(N, C2, H, W)


# speedup vs baseline: 1.5161x; 1.2353x over previous
import functools

import jax
import jax.numpy as jnp
from jax.experimental import pallas as pl
from jax.experimental.pallas import tpu as pltpu

_MB = 4


def _conv_stage_kernel(x_ref, w_ref, b_ref, sc_ref, sh_ref,
                       y_ref, sum_ref, sq_ref, buf_ref,
                       *, H, W, K, transpose_in, affine):
    assert K == 3, "flat-shift tap scheme is written for 3x3"
    MB = x_ref.shape[0]
    HW = H * W
    C = x_ref.shape[1] if transpose_in else x_ref.shape[2]
    S = buf_ref.shape[1]

    col = jax.lax.broadcasted_iota(jnp.int32, (HW, 1), 0) % W
    ml = (col != 0).astype(jnp.bfloat16)
    mr = (col != W - 1).astype(jnp.bfloat16)

    buf_ref[1, 0:W] = jnp.zeros((W, C), jnp.bfloat16)
    buf_ref[1, W + HW:] = jnp.zeros((S - W - HW, C), jnp.bfloat16)
    buf_ref[0, 0:W + 1] = jnp.zeros((W + 1, C), jnp.bfloat16)
    buf_ref[0, W + 1 + HW:] = jnp.zeros((S - W - 1 - HW, C), jnp.bfloat16)
    buf_ref[2, 0:W - 1] = jnp.zeros((W - 1, C), jnp.bfloat16)
    buf_ref[2, W - 1 + HW:] = jnp.zeros((S - W + 1 - HW, C), jnp.bfloat16)

    s_tot = None
    q_tot = None
    for b in range(MB):
        if transpose_in:
            x = jnp.transpose(x_ref[b]).astype(jnp.float32)
        else:
            x = x_ref[b].astype(jnp.float32)
        if affine:
            x = x * sc_ref[...] + sh_ref[...]
        xb = x.astype(jnp.bfloat16)

        buf_ref[1, W:W + HW] = xb
        buf_ref[0, W + 1:W + 1 + HW] = xb * mr
        buf_ref[2, W - 1:W - 1 + HW] = xb * ml

        acc = None
        for kh in range(K):
            for kw in range(K):
                lhs = buf_ref[kw, kh * W:kh * W + HW, :]
                d = jnp.dot(lhs, w_ref[kh * K + kw],
                            preferred_element_type=jnp.float32)
                acc = d if acc is None else acc + d

        y = jnp.maximum(acc + b_ref[...], 0.0)
        s = jnp.sum(y, axis=0, keepdims=True)
        q = jnp.sum(y * y, axis=0, keepdims=True)
        s_tot = s if s_tot is None else s_tot + s
        q_tot = q if q_tot is None else q_tot + q
        y_ref[b] = y.astype(y_ref.dtype)

    sum_ref[0] = s_tot
    sq_ref[0] = q_tot


def _conv_stage(x, w3, b, sc, sh, H, W, transpose_in, affine):
    N = x.shape[0]
    KK, C, Cout = w3.shape
    K = int(round(KK ** 0.5))
    p = (K - 1) // 2
    HW = H * W
    MB = _MB if N % _MB == 0 else 1
    G = N // MB
    S = (H + 2 * p) * W

    if transpose_in:
        x_spec = pl.BlockSpec((MB, C, HW), lambda n: (n, 0, 0))
    else:
        x_spec = pl.BlockSpec((MB, HW, C), lambda n: (n, 0, 0))

    kern = functools.partial(_conv_stage_kernel, H=H, W=W, K=K,
                             transpose_in=transpose_in, affine=affine)
    return pl.pallas_call(
        kern,
        grid=(G,),
        out_shape=(
            jax.ShapeDtypeStruct((N, HW, Cout), jnp.bfloat16),
            jax.ShapeDtypeStruct((G, 1, Cout), jnp.float32),
            jax.ShapeDtypeStruct((G, 1, Cout), jnp.float32),
        ),
        in_specs=[
            x_spec,
            pl.BlockSpec((KK, C, Cout), lambda n: (0, 0, 0)),
            pl.BlockSpec((1, Cout), lambda n: (0, 0)),
            pl.BlockSpec((1, C), lambda n: (0, 0)),
            pl.BlockSpec((1, C), lambda n: (0, 0)),
        ],
        out_specs=(
            pl.BlockSpec((MB, HW, Cout), lambda n: (n, 0, 0)),
            pl.BlockSpec((1, 1, Cout), lambda n: (n, 0, 0)),
            pl.BlockSpec((1, 1, Cout), lambda n: (n, 0, 0)),
        ),
        scratch_shapes=[
            pltpu.VMEM((K, S, C), jnp.bfloat16),
        ],
        compiler_params=pltpu.CompilerParams(
            dimension_semantics=("parallel",),
        ),
    )(x, w3, b, sc, sh)


def _affine_nchw_kernel(y_ref, sc_ref, sh_ref, o_ref):
    for b in range(y_ref.shape[0]):
        y = y_ref[b].astype(jnp.float32) * sc_ref[...] + sh_ref[...]
        o_ref[b] = jnp.transpose(y).astype(o_ref.dtype)


def _apply_affine_nchw(y, sc, sh, out_dtype):
    N, HW, C = y.shape
    MB = _MB if N % _MB == 0 else 1
    G = N // MB
    return pl.pallas_call(
        _affine_nchw_kernel,
        grid=(G,),
        out_shape=jax.ShapeDtypeStruct((N, C, HW), out_dtype),
        in_specs=[
            pl.BlockSpec((MB, HW, C), lambda n: (n, 0, 0)),
            pl.BlockSpec((1, C), lambda n: (0, 0)),
            pl.BlockSpec((1, C), lambda n: (0, 0)),
        ],
        out_specs=pl.BlockSpec((MB, C, HW), lambda n: (n, 0, 0)),
        compiler_params=pltpu.CompilerParams(
            dimension_semantics=("parallel",),
        ),
    )(y, sc, sh)


def _bn_affine(part_sum, part_sq, gamma, beta, count, eps):
    s = jnp.sum(part_sum[:, 0, :], axis=0)
    q = jnp.sum(part_sq[:, 0, :], axis=0)
    mean = s / count
    var = jnp.maximum(q / count - mean * mean, 0.0)
    inv = jax.lax.rsqrt(var + eps)
    scale = gamma.astype(jnp.float32) * inv
    shift = beta.astype(jnp.float32) - mean * scale
    C = scale.shape[0]
    return scale.reshape(1, C), shift.reshape(1, C)


def kernel(x, w1, b1, g1, be1, w2, b2, g2, be2):
    N, Cin, H, W = x.shape
    K = w1.shape[0]
    C1 = w1.shape[3]
    C2 = w2.shape[3]
    eps = 1e-5

    xf = x.reshape(N, Cin, H * W)
    w1b = w1.astype(jnp.bfloat16).reshape(K * K, Cin, C1)
    w2b = w2.astype(jnp.bfloat16).reshape(K * K, C1, C2)
    b1c = b1.astype(jnp.float32).reshape(1, C1)
    b2c = b2.astype(jnp.float32).reshape(1, C2)
    one = jnp.ones((1, Cin), jnp.float32)
    zero = jnp.zeros((1, Cin), jnp.float32)

    y1, s1, q1 = _conv_stage(xf, w1b, b1c, one, zero, H, W,
                             transpose_in=True, affine=False)
    sc1, sh1 = _bn_affine(s1, q1, g1, be1, N * H * W, eps)

    y2, s2, q2 = _conv_stage(y1, w2b, b2c, sc1, sh1, H, W,
                             transpose_in=False, affine=True)
    sc2, sh2 = _bn_affine(s2, q2, g2, be2, N * H * W, eps)

    out = _apply_affine_nchw(y2, sc2, sh2, x.dtype)
    return out.reshape(N, C2, H, W)


# speedup vs baseline: 1.8104x; 1.1942x over previous
import functools

import jax
import jax.numpy as jnp
from jax.experimental import pallas as pl
from jax.experimental.pallas import tpu as pltpu

_MB = 4


def _conv_stage_kernel(x_ref, w_ref, b_ref, sc_ref, sh_ref,
                       y_ref, sum_ref, sq_ref, buf_ref,
                       *, H, W, K, affine):
    assert K == 3, "flat-shift tap scheme is written for 3x3"
    MB = x_ref.shape[0]
    HW = H * W
    C = x_ref.shape[-1]
    S = buf_ref.shape[1]

    col = jax.lax.broadcasted_iota(jnp.int32, (HW, 1), 0) % W
    ml = (col != 0).astype(jnp.bfloat16)
    mr = (col != W - 1).astype(jnp.bfloat16)

    buf_ref[1, 0:W] = jnp.zeros((W, C), jnp.bfloat16)
    buf_ref[1, W + HW:] = jnp.zeros((S - W - HW, C), jnp.bfloat16)
    buf_ref[0, 0:W + 1] = jnp.zeros((W + 1, C), jnp.bfloat16)
    buf_ref[0, W + 1 + HW:] = jnp.zeros((S - W - 1 - HW, C), jnp.bfloat16)
    buf_ref[2, 0:W - 1] = jnp.zeros((W - 1, C), jnp.bfloat16)
    buf_ref[2, W - 1 + HW:] = jnp.zeros((S - W + 1 - HW, C), jnp.bfloat16)

    s_tot = None
    q_tot = None
    for b in range(MB):
        x = x_ref[b].reshape(HW, C).astype(jnp.float32)
        if affine:
            x = x * sc_ref[...] + sh_ref[...]
        xb = x.astype(jnp.bfloat16)

        buf_ref[1, W:W + HW] = xb
        buf_ref[0, W + 1:W + 1 + HW] = xb * mr
        buf_ref[2, W - 1:W - 1 + HW] = xb * ml

        acc = None
        for kh in range(K):
            for kw in range(K):
                lhs = buf_ref[kw, kh * W:kh * W + HW, :]
                d = jnp.dot(lhs, w_ref[kh * K + kw],
                            preferred_element_type=jnp.float32)
                acc = d if acc is None else acc + d

        y = jnp.maximum(acc + b_ref[...], 0.0)
        s = jnp.sum(y, axis=0, keepdims=True)
        q = jnp.sum(y * y, axis=0, keepdims=True)
        s_tot = s if s_tot is None else s_tot + s
        q_tot = q if q_tot is None else q_tot + q
        y_ref[b] = y.astype(y_ref.dtype)

    sum_ref[0] = s_tot
    sq_ref[0] = q_tot


def _conv_stage(x, w3, b, sc, sh, H, W, affine):
    N = x.shape[0]
    KK, C, Cout = w3.shape
    K = int(round(KK ** 0.5))
    p = (K - 1) // 2
    HW = H * W
    MB = _MB if N % _MB == 0 else 1
    G = N // MB
    S = (H + 2 * p) * W

    if x.ndim == 4:
        x_spec = pl.BlockSpec((MB, H, W, C), lambda n: (n, 0, 0, 0))
    else:
        x_spec = pl.BlockSpec((MB, HW, C), lambda n: (n, 0, 0))

    kern = functools.partial(_conv_stage_kernel, H=H, W=W, K=K, affine=affine)
    return pl.pallas_call(
        kern,
        grid=(G,),
        out_shape=(
            jax.ShapeDtypeStruct((N, HW, Cout), jnp.bfloat16),
            jax.ShapeDtypeStruct((G, 1, Cout), jnp.float32),
            jax.ShapeDtypeStruct((G, 1, Cout), jnp.float32),
        ),
        in_specs=[
            x_spec,
            pl.BlockSpec((KK, C, Cout), lambda n: (0, 0, 0)),
            pl.BlockSpec((1, Cout), lambda n: (0, 0)),
            pl.BlockSpec((1, C), lambda n: (0, 0)),
            pl.BlockSpec((1, C), lambda n: (0, 0)),
        ],
        out_specs=(
            pl.BlockSpec((MB, HW, Cout), lambda n: (n, 0, 0)),
            pl.BlockSpec((1, 1, Cout), lambda n: (n, 0, 0)),
            pl.BlockSpec((1, 1, Cout), lambda n: (n, 0, 0)),
        ),
        scratch_shapes=[
            pltpu.VMEM((K, S, C), jnp.bfloat16),
        ],
        compiler_params=pltpu.CompilerParams(
            dimension_semantics=("parallel",),
        ),
    )(x, w3, b, sc, sh)


def _affine_nchw_kernel(y_ref, sc_ref, sh_ref, o_ref):
    for b in range(y_ref.shape[0]):
        y = y_ref[b].astype(jnp.float32) * sc_ref[...] + sh_ref[...]
        o_ref[b] = jnp.transpose(y).astype(o_ref.dtype)


def _apply_affine_nchw(y, sc, sh, out_dtype):
    N, HW, C = y.shape
    MB = _MB if N % _MB == 0 else 1
    G = N // MB
    return pl.pallas_call(
        _affine_nchw_kernel,
        grid=(G,),
        out_shape=jax.ShapeDtypeStruct((N, C, HW), out_dtype),
        in_specs=[
            pl.BlockSpec((MB, HW, C), lambda n: (n, 0, 0)),
            pl.BlockSpec((1, C), lambda n: (0, 0)),
            pl.BlockSpec((1, C), lambda n: (0, 0)),
        ],
        out_specs=pl.BlockSpec((MB, C, HW), lambda n: (n, 0, 0)),
        compiler_params=pltpu.CompilerParams(
            dimension_semantics=("parallel",),
        ),
    )(y, sc, sh)


def _bn_affine(part_sum, part_sq, gamma, beta, count, eps):
    s = jnp.sum(part_sum[:, 0, :], axis=0)
    q = jnp.sum(part_sq[:, 0, :], axis=0)
    mean = s / count
    var = jnp.maximum(q / count - mean * mean, 0.0)
    inv = jax.lax.rsqrt(var + eps)
    scale = gamma.astype(jnp.float32) * inv
    shift = beta.astype(jnp.float32) - mean * scale
    C = scale.shape[0]
    return scale.reshape(1, C), shift.reshape(1, C)


def kernel(x, w1, b1, g1, be1, w2, b2, g2, be2):
    N, Cin, H, W = x.shape
    K = w1.shape[0]
    C1 = w1.shape[3]
    C2 = w2.shape[3]
    eps = 1e-5

    x_nhwc = jnp.transpose(x, (0, 2, 3, 1))
    w1b = w1.astype(jnp.bfloat16).reshape(K * K, Cin, C1)
    w2b = w2.astype(jnp.bfloat16).reshape(K * K, C1, C2)
    b1c = b1.astype(jnp.float32).reshape(1, C1)
    b2c = b2.astype(jnp.float32).reshape(1, C2)
    one = jnp.ones((1, Cin), jnp.float32)
    zero = jnp.zeros((1, Cin), jnp.float32)

    y1, s1, q1 = _conv_stage(x_nhwc, w1b, b1c, one, zero, H, W, affine=False)
    sc1, sh1 = _bn_affine(s1, q1, g1, be1, N * H * W, eps)

    y2, s2, q2 = _conv_stage(y1, w2b, b2c, sc1, sh1, H, W, affine=True)
    sc2, sh2 = _bn_affine(s2, q2, g2, be2, N * H * W, eps)

    out = _apply_affine_nchw(y2, sc2, sh2, x.dtype)
    return out.reshape(N, C2, H, W)


# speedup vs baseline: 1.9432x; 1.0734x over previous
import functools

import jax
import jax.numpy as jnp
from jax.experimental import pallas as pl
from jax.experimental.pallas import tpu as pltpu

_MB = 8


def _conv_stage_kernel(x_ref, w_ref, b_ref, sc_ref, sh_ref,
                       y_ref, sum_ref, sq_ref, buf_ref,
                       *, H, W, K, affine):
    assert K == 3, "flat-shift tap scheme is written for 3x3"
    MB = x_ref.shape[0]
    HW = H * W
    C = x_ref.shape[-1]
    S = buf_ref.shape[1]

    col = jax.lax.broadcasted_iota(jnp.int32, (HW, 1), 0) % W
    ml = (col != 0).astype(jnp.bfloat16)
    mr = (col != W - 1).astype(jnp.bfloat16)

    buf_ref[1, 0:W] = jnp.zeros((W, C), jnp.bfloat16)
    buf_ref[1, W + HW:] = jnp.zeros((S - W - HW, C), jnp.bfloat16)
    buf_ref[0, 0:W + 1] = jnp.zeros((W + 1, C), jnp.bfloat16)
    buf_ref[0, W + 1 + HW:] = jnp.zeros((S - W - 1 - HW, C), jnp.bfloat16)
    buf_ref[2, 0:W - 1] = jnp.zeros((W - 1, C), jnp.bfloat16)
    buf_ref[2, W - 1 + HW:] = jnp.zeros((S - W + 1 - HW, C), jnp.bfloat16)

    s_tot = None
    q_tot = None
    for b in range(MB):
        x = x_ref[b].reshape(HW, C).astype(jnp.float32)
        if affine:
            x = x * sc_ref[...] + sh_ref[...]
        xb = x.astype(jnp.bfloat16)

        buf_ref[1, W:W + HW] = xb
        buf_ref[0, W + 1:W + 1 + HW] = xb * mr
        buf_ref[2, W - 1:W - 1 + HW] = xb * ml

        acc = None
        for kh in range(K):
            for kw in range(K):
                lhs = buf_ref[kw, kh * W:kh * W + HW, :]
                d = jnp.dot(lhs, w_ref[kh * K + kw],
                            preferred_element_type=jnp.float32)
                acc = d if acc is None else acc + d

        y = jnp.maximum(acc + b_ref[...], 0.0)
        s = jnp.sum(y, axis=0, keepdims=True)
        q = jnp.sum(y * y, axis=0, keepdims=True)
        s_tot = s if s_tot is None else s_tot + s
        q_tot = q if q_tot is None else q_tot + q
        y_ref[b] = y.astype(y_ref.dtype)

    sum_ref[0] = s_tot
    sq_ref[0] = q_tot


def _conv_stage(x, w3, b, sc, sh, H, W, affine):
    N = x.shape[0]
    KK, C, Cout = w3.shape
    K = int(round(KK ** 0.5))
    p = (K - 1) // 2
    HW = H * W
    MB = _MB if N % _MB == 0 else 1
    G = N // MB
    S = (H + 2 * p) * W

    if x.ndim == 4:
        x_spec = pl.BlockSpec((MB, H, W, C), lambda n: (n, 0, 0, 0))
    else:
        x_spec = pl.BlockSpec((MB, HW, C), lambda n: (n, 0, 0))

    kern = functools.partial(_conv_stage_kernel, H=H, W=W, K=K, affine=affine)
    return pl.pallas_call(
        kern,
        grid=(G,),
        out_shape=(
            jax.ShapeDtypeStruct((N, HW, Cout), jnp.bfloat16),
            jax.ShapeDtypeStruct((G, 1, Cout), jnp.float32),
            jax.ShapeDtypeStruct((G, 1, Cout), jnp.float32),
        ),
        in_specs=[
            x_spec,
            pl.BlockSpec((KK, C, Cout), lambda n: (0, 0, 0)),
            pl.BlockSpec((1, Cout), lambda n: (0, 0)),
            pl.BlockSpec((1, C), lambda n: (0, 0)),
            pl.BlockSpec((1, C), lambda n: (0, 0)),
        ],
        out_specs=(
            pl.BlockSpec((MB, HW, Cout), lambda n: (n, 0, 0)),
            pl.BlockSpec((1, 1, Cout), lambda n: (n, 0, 0)),
            pl.BlockSpec((1, 1, Cout), lambda n: (n, 0, 0)),
        ),
        scratch_shapes=[
            pltpu.VMEM((K, S, C), jnp.bfloat16),
        ],
        compiler_params=pltpu.CompilerParams(
            dimension_semantics=("parallel",),
        ),
    )(x, w3, b, sc, sh)


def _affine_nchw_kernel(y_ref, sc_ref, sh_ref, o_ref):
    for b in range(y_ref.shape[0]):
        y = y_ref[b].astype(jnp.float32) * sc_ref[...] + sh_ref[...]
        o_ref[b] = jnp.transpose(y).astype(o_ref.dtype)


def _apply_affine_nchw(y, sc, sh, out_dtype):
    N, HW, C = y.shape
    MB = _MB if N % _MB == 0 else 1
    G = N // MB
    return pl.pallas_call(
        _affine_nchw_kernel,
        grid=(G,),
        out_shape=jax.ShapeDtypeStruct((N, C, HW), out_dtype),
        in_specs=[
            pl.BlockSpec((MB, HW, C), lambda n: (n, 0, 0)),
            pl.BlockSpec((1, C), lambda n: (0, 0)),
            pl.BlockSpec((1, C), lambda n: (0, 0)),
        ],
        out_specs=pl.BlockSpec((MB, C, HW), lambda n: (n, 0, 0)),
        compiler_params=pltpu.CompilerParams(
            dimension_semantics=("parallel",),
        ),
    )(y, sc, sh)


def _bn_affine(part_sum, part_sq, gamma, beta, count, eps):
    s = jnp.sum(part_sum[:, 0, :], axis=0)
    q = jnp.sum(part_sq[:, 0, :], axis=0)
    mean = s / count
    var = jnp.maximum(q / count - mean * mean, 0.0)
    inv = jax.lax.rsqrt(var + eps)
    scale = gamma.astype(jnp.float32) * inv
    shift = beta.astype(jnp.float32) - mean * scale
    C = scale.shape[0]
    return scale.reshape(1, C), shift.reshape(1, C)


def kernel(x, w1, b1, g1, be1, w2, b2, g2, be2):
    N, Cin, H, W = x.shape
    K = w1.shape[0]
    C1 = w1.shape[3]
    C2 = w2.shape[3]
    eps = 1e-5

    x_nhwc = jnp.transpose(x, (0, 2, 3, 1))
    w1b = w1.astype(jnp.bfloat16).reshape(K * K, Cin, C1)
    w2b = w2.astype(jnp.bfloat16).reshape(K * K, C1, C2)
    b1c = b1.astype(jnp.float32).reshape(1, C1)
    b2c = b2.astype(jnp.float32).reshape(1, C2)
    one = jnp.ones((1, Cin), jnp.float32)
    zero = jnp.zeros((1, Cin), jnp.float32)

    y1, s1, q1 = _conv_stage(x_nhwc, w1b, b1c, one, zero, H, W, affine=False)
    sc1, sh1 = _bn_affine(s1, q1, g1, be1, N * H * W, eps)

    y2, s2, q2 = _conv_stage(y1, w2b, b2c, sc1, sh1, H, W, affine=True)
    sc2, sh2 = _bn_affine(s2, q2, g2, be2, N * H * W, eps)

    out = _apply_affine_nchw(y2, sc2, sh2, x.dtype)
    return out.reshape(N, C2, H, W)


# speedup vs baseline: 2.0091x; 1.0339x over previous
import functools

import jax
import jax.numpy as jnp
from jax.experimental import pallas as pl
from jax.experimental.pallas import tpu as pltpu

_MB = 16


def _conv_stage_kernel(x_ref, w_ref, b_ref, sc_ref, sh_ref,
                       y_ref, sum_ref, sq_ref, buf_ref,
                       *, H, W, K, affine):
    assert K == 3, "flat-shift tap scheme is written for 3x3"
    MB = x_ref.shape[0]
    HW = H * W
    C = x_ref.shape[-1]
    S = buf_ref.shape[1]

    col = jax.lax.broadcasted_iota(jnp.int32, (HW, 1), 0) % W
    ml = (col != 0).astype(jnp.bfloat16)
    mr = (col != W - 1).astype(jnp.bfloat16)

    buf_ref[1, 0:W] = jnp.zeros((W, C), jnp.bfloat16)
    buf_ref[1, W + HW:] = jnp.zeros((S - W - HW, C), jnp.bfloat16)
    buf_ref[0, 0:W + 1] = jnp.zeros((W + 1, C), jnp.bfloat16)
    buf_ref[0, W + 1 + HW:] = jnp.zeros((S - W - 1 - HW, C), jnp.bfloat16)
    buf_ref[2, 0:W - 1] = jnp.zeros((W - 1, C), jnp.bfloat16)
    buf_ref[2, W - 1 + HW:] = jnp.zeros((S - W + 1 - HW, C), jnp.bfloat16)

    s_tot = None
    q_tot = None
    for b in range(MB):
        x = x_ref[b].reshape(HW, C).astype(jnp.float32)
        if affine:
            x = x * sc_ref[...] + sh_ref[...]
        xb = x.astype(jnp.bfloat16)

        buf_ref[1, W:W + HW] = xb
        buf_ref[0, W + 1:W + 1 + HW] = xb * mr
        buf_ref[2, W - 1:W - 1 + HW] = xb * ml

        acc = None
        for kh in range(K):
            for kw in range(K):
                lhs = buf_ref[kw, kh * W:kh * W + HW, :]
                d = jnp.dot(lhs, w_ref[kh * K + kw],
                            preferred_element_type=jnp.float32)
                acc = d if acc is None else acc + d

        y = jnp.maximum(acc + b_ref[...], 0.0)
        s = jnp.sum(y, axis=0, keepdims=True)
        q = jnp.sum(y * y, axis=0, keepdims=True)
        s_tot = s if s_tot is None else s_tot + s
        q_tot = q if q_tot is None else q_tot + q
        y_ref[b] = y.astype(y_ref.dtype)

    sum_ref[0] = s_tot
    sq_ref[0] = q_tot


def _conv_stage(x, w3, b, sc, sh, H, W, affine):
    N = x.shape[0]
    KK, C, Cout = w3.shape
    K = int(round(KK ** 0.5))
    p = (K - 1) // 2
    HW = H * W
    MB = _MB if N % _MB == 0 else 1
    G = N // MB
    S = (H + 2 * p) * W

    if x.ndim == 4:
        x_spec = pl.BlockSpec((MB, H, W, C), lambda n: (n, 0, 0, 0))
    else:
        x_spec = pl.BlockSpec((MB, HW, C), lambda n: (n, 0, 0))

    kern = functools.partial(_conv_stage_kernel, H=H, W=W, K=K, affine=affine)
    return pl.pallas_call(
        kern,
        grid=(G,),
        out_shape=(
            jax.ShapeDtypeStruct((N, HW, Cout), jnp.bfloat16),
            jax.ShapeDtypeStruct((G, 1, Cout), jnp.float32),
            jax.ShapeDtypeStruct((G, 1, Cout), jnp.float32),
        ),
        in_specs=[
            x_spec,
            pl.BlockSpec((KK, C, Cout), lambda n: (0, 0, 0)),
            pl.BlockSpec((1, Cout), lambda n: (0, 0)),
            pl.BlockSpec((1, C), lambda n: (0, 0)),
            pl.BlockSpec((1, C), lambda n: (0, 0)),
        ],
        out_specs=(
            pl.BlockSpec((MB, HW, Cout), lambda n: (n, 0, 0)),
            pl.BlockSpec((1, 1, Cout), lambda n: (n, 0, 0)),
            pl.BlockSpec((1, 1, Cout), lambda n: (n, 0, 0)),
        ),
        scratch_shapes=[
            pltpu.VMEM((K, S, C), jnp.bfloat16),
        ],
        compiler_params=pltpu.CompilerParams(
            dimension_semantics=("parallel",),
            vmem_limit_bytes=48 * 1024 * 1024,
        ),
    )(x, w3, b, sc, sh)


def _affine_nchw_kernel(y_ref, sc_ref, sh_ref, o_ref):
    for b in range(y_ref.shape[0]):
        y = y_ref[b].astype(jnp.float32) * sc_ref[...] + sh_ref[...]
        o_ref[b] = jnp.transpose(y).astype(o_ref.dtype)


def _apply_affine_nchw(y, sc, sh, out_dtype):
    N, HW, C = y.shape
    MB = _MB if N % _MB == 0 else 1
    G = N // MB
    return pl.pallas_call(
        _affine_nchw_kernel,
        grid=(G,),
        out_shape=jax.ShapeDtypeStruct((N, C, HW), out_dtype),
        in_specs=[
            pl.BlockSpec((MB, HW, C), lambda n: (n, 0, 0)),
            pl.BlockSpec((1, C), lambda n: (0, 0)),
            pl.BlockSpec((1, C), lambda n: (0, 0)),
        ],
        out_specs=pl.BlockSpec((MB, C, HW), lambda n: (n, 0, 0)),
        compiler_params=pltpu.CompilerParams(
            dimension_semantics=("parallel",),
        ),
    )(y, sc, sh)


def _bn_affine(part_sum, part_sq, gamma, beta, count, eps):
    s = jnp.sum(part_sum[:, 0, :], axis=0)
    q = jnp.sum(part_sq[:, 0, :], axis=0)
    mean = s / count
    var = jnp.maximum(q / count - mean * mean, 0.0)
    inv = jax.lax.rsqrt(var + eps)
    scale = gamma.astype(jnp.float32) * inv
    shift = beta.astype(jnp.float32) - mean * scale
    C = scale.shape[0]
    return scale.reshape(1, C), shift.reshape(1, C)


def kernel(x, w1, b1, g1, be1, w2, b2, g2, be2):
    N, Cin, H, W = x.shape
    K = w1.shape[0]
    C1 = w1.shape[3]
    C2 = w2.shape[3]
    eps = 1e-5

    x_nhwc = jnp.transpose(x, (0, 2, 3, 1))
    w1b = w1.astype(jnp.bfloat16).reshape(K * K, Cin, C1)
    w2b = w2.astype(jnp.bfloat16).reshape(K * K, C1, C2)
    b1c = b1.astype(jnp.float32).reshape(1, C1)
    b2c = b2.astype(jnp.float32).reshape(1, C2)
    one = jnp.ones((1, Cin), jnp.float32)
    zero = jnp.zeros((1, Cin), jnp.float32)

    y1, s1, q1 = _conv_stage(x_nhwc, w1b, b1c, one, zero, H, W, affine=False)
    sc1, sh1 = _bn_affine(s1, q1, g1, be1, N * H * W, eps)

    y2, s2, q2 = _conv_stage(y1, w2b, b2c, sc1, sh1, H, W, affine=True)
    sc2, sh2 = _bn_affine(s2, q2, g2, be2, N * H * W, eps)

    out = _apply_affine_nchw(y2, sc2, sh2, x.dtype)
    return out.reshape(N, C2, H, W)
